# Initial kernel scaffold; baseline (speedup 1.0000x reference)
import jax
import jax.numpy as jnp
from jax.experimental import pallas as pl


def kernel(_xyz, _scaling, _rotation, _features_dc, _normf, _opacity):
    raise NotImplementedError("write your pallas kernel here")


# speedup vs baseline: 1.0720x; 1.0720x over previous
import math

import jax
import jax.numpy as jnp
from jax.experimental import pallas as pl
from jax.experimental.pallas import tpu as pltpu

H = 256
W = 256
HW = H * W
N_RAW = 10000
CHUNK = 128
NPAD = 10240
NCHUNKS = NPAD // CHUNK
PXT = 8192
NTILES = HW // PXT


def _raster_kernel(xyz_ref, sc_ref, rot_ref, fdc_ref, nf_ref, op_ref,
                   gx_ref, gy_ref, out_ref):
    j = pl.program_id(1)

    xy = jnp.tanh(xyz_ref[...])
    scaling = jnp.abs(sc_ref[...] + 0.5)
    theta = jax.nn.sigmoid(rot_ref[...]) * (2.0 * math.pi)
    normf = jnp.exp(nf_ref[...])
    amp = fdc_ref[...] * op_ref[...] * (normf[0:1] * normf[1:2])
    c = jnp.cos(theta)
    s = jnp.sin(theta)
    sx2 = scaling[0:1] ** 2 + 1e-8
    sy2 = scaling[1:2] ** 2 + 1e-8
    covA = c * c * sx2 + s * s * sy2
    covB = c * s * (sx2 - sy2)
    covC = s * s * sx2 + c * c * sy2
    det = covA * covC - covB * covB + 1e-12
    Ah = -0.5 * covC / det
    Bh = covB / det
    Ch = -0.5 * covA / det

    pxc = xy[0:1].reshape(CHUNK, 1)
    pyc = xy[1:2].reshape(CHUNK, 1)
    Ac = Ah.reshape(CHUNK, 1)
    Bc = Bh.reshape(CHUNK, 1)
    Cc = Ch.reshape(CHUNK, 1)
    ampb = amp.astype(jnp.bfloat16)

    gx = gx_ref[...]
    gy = gy_ref[...]
    dx = gx - pxc
    dy = gy - pyc
    q = dx * (Ac * dx + Bc * dy) + Cc * (dy * dy)
    wb = jnp.exp(q).astype(jnp.bfloat16)
    contrib = jax.lax.dot_general(
        ampb, wb, (((1,), (0,)), ((), ())),
        preferred_element_type=jnp.float32)

    @pl.when(j == 0)
    def _init():
        out_ref[...] = contrib

    @pl.when(j > 0)
    def _acc():
        out_ref[...] += contrib

    @pl.when(j == NCHUNKS - 1)
    def _finish():
        out_ref[...] = jnp.clip(out_ref[...], 0.0, 1.0)


def kernel(_xyz, _scaling, _rotation, _features_dc, _normf, _opacity):
    f32 = jnp.float32
    pad = ((0, NPAD - N_RAW), (0, 0))
    xyzT = jnp.pad(_xyz.astype(f32), pad).T
    scT = jnp.pad(_scaling.astype(f32), pad).T
    rotT = jnp.pad(_rotation.astype(f32), pad).T
    fdcT = jnp.pad(_features_dc.astype(f32), pad).T
    nfT = jnp.pad(_normf.astype(f32), pad).T
    opT = jnp.pad(_opacity.astype(f32), pad).T

    gx = (jnp.arange(W, dtype=f32) + 0.5) / W * 2.0 - 1.0
    gy = (jnp.arange(H, dtype=f32) + 0.5) / H * 2.0 - 1.0
    GX, GY = jnp.meshgrid(gx, gy)
    gxf = GX.reshape(1, HW)
    gyf = GY.reshape(1, HW)

    pt_spec = lambda k: pl.BlockSpec((k, CHUNK), lambda i, j: (0, j))
    px_spec = pl.BlockSpec((1, PXT), lambda i, j: (0, i))

    out = pl.pallas_call(
        _raster_kernel,
        grid=(NTILES, NCHUNKS),
        in_specs=[pt_spec(2), pt_spec(2), pt_spec(1), pt_spec(3),
                  pt_spec(2), pt_spec(1), px_spec, px_spec],
        out_specs=pl.BlockSpec((3, PXT), lambda i, j: (0, i)),
        out_shape=jax.ShapeDtypeStruct((3, HW), f32),
        compiler_params=pltpu.CompilerParams(
            dimension_semantics=("arbitrary", "arbitrary"),
        ),
    )(xyzT, scT, rotT, fdcT, nfT, opT, gxf, gyf)

    return out.reshape(1, 3, H, W)


# speedup vs baseline: 3.3556x; 3.1301x over previous
import math

import jax
import jax.numpy as jnp
from jax.experimental import pallas as pl
from jax.experimental.pallas import tpu as pltpu

H = 256
W = 256
HW = H * W
N_RAW = 10000
CHUNK = 128
NPAD = 10240
NCHUNKS = NPAD // CHUNK
HALF = 128
NHALF = W // HALF
PIX_STEP = 2.0 / H
GY0 = -1.0 + 0.5 * PIX_STEP


def _raster_kernel(xyz_ref, sc_ref, rot_ref, fdc_ref, nf_ref, op_ref,
                   gx_ref, out_ref):
    j = pl.program_id(0)

    xy = jnp.tanh(xyz_ref[...])
    scaling = jnp.abs(sc_ref[...] + 0.5)
    theta = jax.nn.sigmoid(rot_ref[...]) * (2.0 * math.pi)
    normf = jnp.exp(nf_ref[...])
    amp = fdc_ref[...] * op_ref[...] * (normf[0:1] * normf[1:2])
    c = jnp.cos(theta)
    s = jnp.sin(theta)
    sx2 = scaling[0:1] ** 2 + 1e-8
    sy2 = scaling[1:2] ** 2 + 1e-8
    covA = c * c * sx2 + s * s * sy2
    covB = c * s * (sx2 - sy2)
    covC = s * s * sx2 + c * c * sy2
    det = covA * covC - covB * covB + 1e-12
    LOG2E = math.log2(math.e)
    Ah = (-0.5 * LOG2E) * covC / det
    Bh = LOG2E * covB / det
    Ch = (-0.5 * LOG2E) * covA / det

    pxc = xy[0:1].reshape(CHUNK, 1)
    pyc = xy[1:2].reshape(CHUNK, 1)
    Ac = Ah.reshape(CHUNK, 1)
    Bc = Bh.reshape(CHUNK, 1)
    Cc = Ch.reshape(CHUNK, 1)
    ampb = amp.astype(jnp.bfloat16)

    gx = gx_ref[...]
    dx = gx - pxc
    u0 = (Ac * dx - Bc * pyc) * dx + Cc * (pyc * pyc)
    u1 = Bc * dx - (2.0 * Cc * pyc)
    u2 = Cc + 0.0 * dx

    k = PIX_STEP
    d0 = (k * u1 + (k * k) * u2) + (2.0 * k * GY0) * u2
    rho = jnp.exp2((2.0 * k * k) * u2)

    @pl.when(j == 0)
    def _init():
        out_ref[...] = jnp.zeros((3, HW), jnp.float32)

    for h in range(NHALF):
        hs = slice(h * HALF, (h + 1) * HALF)
        w = jnp.exp2(u0[:, hs] + GY0 * (u1[:, hs] + GY0 * u2[:, hs]))
        m = jnp.exp2(d0[:, hs])
        rh = rho[:, hs]
        for y in range(H):
            wb = w.astype(jnp.bfloat16)
            contrib = jax.lax.dot_general(
                ampb, wb, (((1,), (0,)), ((), ())),
                preferred_element_type=jnp.float32)
            col = y * W + h * HALF
            out_ref[:, col:col + HALF] += contrib
            if y + 1 < H:
                w = w * m
                m = m * rh

    @pl.when(j == NCHUNKS - 1)
    def _finish():
        out_ref[...] = jnp.clip(out_ref[...], 0.0, 1.0)


def kernel(_xyz, _scaling, _rotation, _features_dc, _normf, _opacity):
    f32 = jnp.float32
    pad = ((0, NPAD - N_RAW), (0, 0))
    xyzT = jnp.pad(_xyz.astype(f32), pad).T
    scT = jnp.pad(_scaling.astype(f32), pad).T
    rotT = jnp.pad(_rotation.astype(f32), pad).T
    fdcT = jnp.pad(_features_dc.astype(f32), pad).T
    nfT = jnp.pad(_normf.astype(f32), pad).T
    opT = jnp.pad(_opacity.astype(f32), pad).T

    gx = ((jnp.arange(W, dtype=f32) + 0.5) / W * 2.0 - 1.0).reshape(1, W)

    pt_spec = lambda k: pl.BlockSpec((k, CHUNK), lambda j: (0, j))

    out = pl.pallas_call(
        _raster_kernel,
        grid=(NCHUNKS,),
        in_specs=[pt_spec(2), pt_spec(2), pt_spec(1), pt_spec(3),
                  pt_spec(2), pt_spec(1),
                  pl.BlockSpec((1, W), lambda j: (0, 0))],
        out_specs=pl.BlockSpec((3, HW), lambda j: (0, 0)),
        out_shape=jax.ShapeDtypeStruct((3, HW), f32),
        compiler_params=pltpu.CompilerParams(
            dimension_semantics=("arbitrary",),
        ),
    )(xyzT, scT, rotT, fdcT, nfT, opT, gx)

    return out.reshape(1, 3, H, W)


# speedup vs baseline: 4.7138x; 1.4048x over previous
import math

import jax
import jax.numpy as jnp
from jax.experimental import pallas as pl
from jax.experimental.pallas import tpu as pltpu

H = 256
W = 256
HW = H * W
N_RAW = 10000
CHUNK = 128
NPAD = 10240
NCHUNKS = NPAD // CHUNK
HALF = 128
NHALF = W // HALF
PIX_STEP = 2.0 / H
GY0 = -1.0 + 0.5 * PIX_STEP


def _raster_kernel(xyz_ref, sc_ref, rot_ref, fdc_ref, nf_ref, op_ref,
                   gx_ref, out_ref):
    j = pl.program_id(0)

    xy = jnp.tanh(xyz_ref[...])
    scaling = jnp.abs(sc_ref[...] + 0.5)
    theta = jax.nn.sigmoid(rot_ref[...]) * (2.0 * math.pi)
    normf = jnp.exp(nf_ref[...])
    amp = fdc_ref[...] * op_ref[...] * (normf[0:1] * normf[1:2])
    c = jnp.cos(theta)
    s = jnp.sin(theta)
    sx2 = scaling[0:1] ** 2 + 1e-8
    sy2 = scaling[1:2] ** 2 + 1e-8
    covA = c * c * sx2 + s * s * sy2
    covB = c * s * (sx2 - sy2)
    covC = s * s * sx2 + c * c * sy2
    det = covA * covC - covB * covB + 1e-12
    LOG2E = math.log2(math.e)
    Ah = (-0.5 * LOG2E) * covC / det
    Bh = LOG2E * covB / det
    Ch = (-0.5 * LOG2E) * covA / det

    pxc = xy[0:1].reshape(CHUNK, 1)
    pyc = xy[1:2].reshape(CHUNK, 1)
    Ac = Ah.reshape(CHUNK, 1)
    Bc = Bh.reshape(CHUNK, 1)
    Cc = Ch.reshape(CHUNK, 1)
    ampb = amp.astype(jnp.bfloat16)

    gx = gx_ref[...]
    dx = gx - pxc
    u0 = (Ac * dx - Bc * pyc) * dx + Cc * (pyc * pyc)
    u1 = Bc * dx - (2.0 * Cc * pyc)
    u2 = Cc + 0.0 * dx

    k = PIX_STEP
    d0 = (k * u1 + (k * k) * u2) + (2.0 * k * GY0) * u2
    rho = jnp.exp2((2.0 * k * k) * u2)

    @pl.when(j == 0)
    def _init():
        out_ref[...] = jnp.zeros((3, HW), jnp.float32)

    for h in range(NHALF):
        hs = slice(h * HALF, (h + 1) * HALF)
        w = jnp.exp2(u0[:, hs] + GY0 * (u1[:, hs] + GY0 * u2[:, hs]))
        m = jnp.exp2(d0[:, hs])
        rh = rho[:, hs]
        for g in range(H // 8):
            rows = []
            for r in range(8):
                rows.append(w.astype(jnp.bfloat16))
                if g * 8 + r + 1 < H:
                    w = w * m
                    m = m * rh
            wcat = jnp.concatenate(rows, axis=1)
            contrib = jax.lax.dot_general(
                ampb, wcat, (((1,), (0,)), ((), ())),
                preferred_element_type=jnp.float32)
            for r in range(8):
                col = (g * 8 + r) * W + h * HALF
                out_ref[:, col:col + HALF] += contrib[:, r * HALF:(r + 1) * HALF]

    @pl.when(j == NCHUNKS - 1)
    def _finish():
        out_ref[...] = jnp.clip(out_ref[...], 0.0, 1.0)


def kernel(_xyz, _scaling, _rotation, _features_dc, _normf, _opacity):
    f32 = jnp.float32
    pad = ((0, NPAD - N_RAW), (0, 0))
    xyzT = jnp.pad(_xyz.astype(f32), pad).T
    scT = jnp.pad(_scaling.astype(f32), pad).T
    rotT = jnp.pad(_rotation.astype(f32), pad).T
    fdcT = jnp.pad(_features_dc.astype(f32), pad).T
    nfT = jnp.pad(_normf.astype(f32), pad).T
    opT = jnp.pad(_opacity.astype(f32), pad).T

    gx = ((jnp.arange(W, dtype=f32) + 0.5) / W * 2.0 - 1.0).reshape(1, W)

    pt_spec = lambda k: pl.BlockSpec((k, CHUNK), lambda j: (0, j))

    out = pl.pallas_call(
        _raster_kernel,
        grid=(NCHUNKS,),
        in_specs=[pt_spec(2), pt_spec(2), pt_spec(1), pt_spec(3),
                  pt_spec(2), pt_spec(1),
                  pl.BlockSpec((1, W), lambda j: (0, 0))],
        out_specs=pl.BlockSpec((3, HW), lambda j: (0, 0)),
        out_shape=jax.ShapeDtypeStruct((3, HW), f32),
        compiler_params=pltpu.CompilerParams(
            dimension_semantics=("arbitrary",),
        ),
    )(xyzT, scT, rotT, fdcT, nfT, opT, gx)

    return out.reshape(1, 3, H, W)
